# baseline (speedup 1.0000x reference)
import functools

import jax
import jax.numpy as jnp
from jax.experimental import pallas as pl
from jax.experimental.pallas import tpu as pltpu


_BS = 8
_N = 2048
_NC = 32
_ROWS = 2048
_NSTREAMS = 4
_W = _N // _NSTREAMS


def _mse_kernel(*refs):
    x_refs = refs[:_NSTREAMS]
    starts_ref, ends_ref, out_ref, acc_ref = refs[_NSTREAMS:]
    b = pl.program_id(0)

    @pl.when(b == 0)
    def _init():
        acc_ref[...] = jnp.zeros_like(acc_ref)

    starts = starts_ref[0, 0, :].reshape(1, _NC)
    ends = ends_ref[0, 0, :].reshape(1, _NC)

    rows = jax.lax.broadcasted_iota(jnp.int32, (_ROWS, _NC), 0)
    inb = (rows >= starts) & (rows < ends)
    lo = jnp.min(jnp.where(inb, starts, _N), axis=1, keepdims=True)
    hi = jnp.max(jnp.where(inb, ends, 0), axis=1, keepdims=True)

    ones = jnp.ones((8, _ROWS), jnp.float32)

    def stream(x_ref, base):
        cols = jax.lax.broadcasted_iota(jnp.int32, (_ROWS, _W), 1) + base
        rel = jax.lax.bitcast_convert_type(cols - lo, jnp.uint32)
        width = jax.lax.bitcast_convert_type(hi - lo, jnp.uint32)
        pred = rel < width
        x = x_ref[0]
        diff = jnp.where(pred, x - 1.0, x)
        d2 = diff * diff
        return jax.lax.dot_general(
            ones, d2, (((1,), (0,)), ((), ())), preferred_element_type=jnp.float32
        )

    for s in range(_NSTREAMS):
        acc_ref[:, s * _W : (s + 1) * _W] += stream(x_refs[s], s * _W)

    @pl.when(b == _BS - 1)
    def _fin():
        out_ref[...] = jnp.sum(acc_ref[...]).reshape(1, 1)


@functools.partial(jax.jit, static_argnames=())
def _loss(raw_scores, starts, ends):
    def make_spec(s):
        return pl.BlockSpec((1, _ROWS, _W), lambda b, s=s: (b, 0, s))

    total = pl.pallas_call(
        _mse_kernel,
        grid=(_BS,),
        in_specs=[make_spec(s) for s in range(_NSTREAMS)]
        + [
            pl.BlockSpec((1, 1, _NC), lambda b: (b, 0, 0)),
            pl.BlockSpec((1, 1, _NC), lambda b: (b, 0, 0)),
        ],
        out_specs=pl.BlockSpec((1, 1), lambda b: (0, 0)),
        out_shape=jax.ShapeDtypeStruct((1, 1), jnp.float32),
        scratch_shapes=[pltpu.VMEM((8, _N), jnp.float32)],
    )(*([raw_scores] * _NSTREAMS), starts, ends)
    return total[0, 0] / jnp.float32(_BS * _N * _N * 8)


def kernel(raw_scores, cluster_sizes):
    cs = cluster_sizes.astype(jnp.int32)
    starts = jnp.concatenate(
        [jnp.zeros((_BS, 1), dtype=jnp.int32), cs[:, :-1]], axis=1
    ).reshape(_BS, 1, _NC)
    ends = starts + cs.reshape(_BS, 1, _NC)
    return _loss(raw_scores, starts, ends)
